# baseline (speedup 1.0000x reference)
import functools

import jax
import jax.numpy as jnp
from jax import lax
from jax.experimental import pallas as pl
from jax.experimental.pallas import tpu as pltpu
from jax.experimental.pallas import tpu_sc as plsc

NC = 2
NS = 16
L = 16
CH = 128
AHEAD = 2
NBUF = 4


def _mesh():
    return plsc.VectorSubcoreMesh(core_axis_name="c", subcore_axis_name="s")


_SC_PARAMS = pltpu.CompilerParams(use_tc_tiling_on_sc=False)


def _pad_nodes(n_nodes):
    per = -(-n_nodes // (NS * 40)) * 40
    return per * NS


def _zero_acc_slice(zb, acc, sid, n_rows_per_subcore):
    zr, d = zb.shape

    @pl.loop(0, zr)
    def _(r):
        for f in range(d // L):
            zb[r, pl.ds(f * L, L)] = jnp.zeros((L,), jnp.float32)

    @pl.loop(0, n_rows_per_subcore // zr)
    def _(z):
        pltpu.sync_copy(zb, acc.at[pl.ds(sid * n_rows_per_subcore + z * zr, zr)])


def _sc_degree(col3, ewb, n_nodes):
    c_chunks = col3.shape[1]
    c_half = c_chunks // NC
    n_pad = _pad_nodes(n_nodes)
    rps = n_pad // NS

    @functools.partial(
        pl.kernel,
        out_type=jax.ShapeDtypeStruct((NC, n_pad, L), jnp.float32),
        mesh=_mesh(),
        compiler_params=_SC_PARAMS,
        scratch_types=[
            pltpu.VMEM((c_chunks, CH), jnp.int32),
            pltpu.VMEM((NBUF, CH, L), jnp.float32),
            pltpu.VMEM_SHARED((n_pad, L), jnp.float32),
            pltpu.SemaphoreType.DMA((NBUF,)),
            pltpu.SemaphoreType.DMA((NBUF,)),
        ],
    )
    def k(col_hbm, ewb_hbm, out_hbm, col_v, ewb_v, acc, esem, ssem):
        cid = lax.axis_index("c")
        sid = lax.axis_index("s")
        base = cid * c_half
        pltpu.sync_copy(col_hbm.at[sid], col_v)
        _zero_acc_slice(ewb_v.at[0], acc, sid, rps)
        plsc.subcore_barrier()

        def issue(ci, b):
            pltpu.async_copy(ewb_hbm.at[sid, base + ci], ewb_v.at[b],
                             esem.at[b])

        def wait_scatter(b):
            pltpu.make_async_copy(ewb_v.at[b], acc.at[col_v.at[0]],
                                  ssem.at[b]).wait()

        issue(0, 0)
        issue(1, 1)

        @pl.loop(0, c_half // NBUF)
        def _(c4):
            for kk in range(NBUF):
                b = kk
                f = (kk + AHEAD) % NBUF
                c = c4 * NBUF + kk

                @pl.when(c + AHEAD < c_half)
                def _():
                    @pl.when(c + AHEAD >= NBUF)
                    def _():
                        wait_scatter(f)

                    issue(c + AHEAD, f)

                pltpu.make_async_copy(ewb_hbm.at[sid, 0], ewb_v.at[b],
                                      esem.at[b]).wait()
                pltpu.async_copy(ewb_v.at[b], acc.at[col_v.at[base + c]],
                                 ssem.at[b], add=True)

        for b in range(NBUF):
            wait_scatter(b)

        plsc.subcore_barrier()
        pltpu.sync_copy(acc.at[pl.ds(sid * rps, rps)],
                        out_hbm.at[cid, pl.ds(sid * rps, rps)])

    return k(col3, ewb)


def _sc_scatter(gh, row3, col3, ewb, n_nodes):
    dh = gh.shape[2]
    c_chunks = row3.shape[1]
    n_pad = _pad_nodes(n_nodes)
    rps = n_pad // NS
    fd = dh // L

    @functools.partial(
        pl.kernel,
        out_type=jax.ShapeDtypeStruct((NC, n_pad, dh), jnp.float32),
        mesh=_mesh(),
        compiler_params=_SC_PARAMS,
        scratch_types=[
            pltpu.VMEM((c_chunks, CH), jnp.int32),
            pltpu.VMEM((c_chunks, CH), jnp.int32),
            pltpu.VMEM((NBUF, CH, L), jnp.float32),
            pltpu.VMEM((NBUF, CH, dh), jnp.float32),
            pltpu.VMEM_SHARED((n_pad, dh), jnp.float32),
            pltpu.SemaphoreType.DMA((NBUF,)),
            pltpu.SemaphoreType.DMA((NBUF,)),
            pltpu.SemaphoreType.DMA((NBUF,)),
        ],
    )
    def k(gh_hbm, row_hbm, col_hbm, ewb_hbm, out_hbm,
          row_v, col_v, ewb_v, gbuf, acc, gsem, esem, ssem):
        cid = lax.axis_index("c")
        sid = lax.axis_index("s")
        g_src = gh_hbm.at[cid]

        pltpu.sync_copy(row_hbm.at[sid], row_v)
        pltpu.sync_copy(col_hbm.at[sid], col_v)
        _zero_acc_slice(gbuf.at[0], acc, sid, rps)
        plsc.subcore_barrier()

        def issue(ci, b):
            pltpu.async_copy(ewb_hbm.at[sid, ci], ewb_v.at[b], esem.at[b])
            pltpu.async_copy(g_src.at[row_v.at[ci]], gbuf.at[b], gsem.at[b])

        def wait_scatter(b):
            pltpu.make_async_copy(gbuf.at[b], acc.at[col_v.at[0]],
                                  ssem.at[b]).wait()

        issue(0, 0)
        issue(1, 1)

        @pl.loop(0, c_chunks // NBUF)
        def _(c4):
            for kk in range(NBUF):
                b = kk
                f = (kk + AHEAD) % NBUF
                c = c4 * NBUF + kk

                @pl.when(c + AHEAD < c_chunks)
                def _():
                    @pl.when(c + AHEAD >= NBUF)
                    def _():
                        wait_scatter(f)

                    issue(c + AHEAD, f)

                pltpu.make_async_copy(g_src.at[row_v.at[0]], gbuf.at[b],
                                      gsem.at[b]).wait()
                pltpu.make_async_copy(ewb_hbm.at[sid, 0], ewb_v.at[b],
                                      esem.at[b]).wait()

                eb = ewb_v.at[b]
                gb = gbuf.at[b]

                @plsc.parallel_loop(0, CH, step=1, unroll=16)
                def _(j):
                    s = eb[j, :]
                    for ff in range(fd):
                        sl = pl.ds(ff * L, L)
                        gb[j, sl] = gb[j, sl] * s

                pltpu.async_copy(gb, acc.at[col_v.at[c]], ssem.at[b],
                                 add=True)

        for b in range(NBUF):
            wait_scatter(b)

        plsc.subcore_barrier()
        pltpu.sync_copy(acc.at[pl.ds(sid * rps, rps)],
                        out_hbm.at[cid, pl.ds(sid * rps, rps)])

    return k(gh, row3, col3, ewb)


def _dis_from(degp_blk):
    deg = degp_blk[0, :, 0] + degp_blk[1, :, 0] + 1.0
    return jnp.where(deg > 0, lax.rsqrt(jnp.maximum(deg, 1e-30)), 0.0)


def _split(g):
    dh = g.shape[1] // 2
    return jnp.stack([g[:, :dh], g[:, dh:]], axis=0)


_B = 1000


def _tc_pre(degp, x, w):
    n, d_in = x.shape
    dh = w.shape[1] // 2

    def body(degp_ref, x_ref, w_ref, g_ref):
        dis = _dis_from(degp_ref[...])
        h = jnp.dot(x_ref[...], w_ref[...], preferred_element_type=jnp.float32)
        g_ref[...] = _split(dis[:, None] * h)

    return pl.pallas_call(
        body,
        grid=(n // _B,),
        in_specs=[
            pl.BlockSpec((NC, _B, L), lambda i: (0, i, 0)),
            pl.BlockSpec((_B, d_in), lambda i: (i, 0)),
            pl.BlockSpec((d_in, 2 * dh), lambda i: (0, 0)),
        ],
        out_specs=pl.BlockSpec((NC, _B, dh), lambda i: (0, i, 0)),
        out_shape=jax.ShapeDtypeStruct((NC, n, dh), jnp.float32),
    )(degp, x, w)


def _tc_mid(degp, p, gh, b, w):
    n = gh.shape[1]
    dh = gh.shape[2]
    d = 2 * dh
    dhn = w.shape[1] // 2

    def body(degp_ref, p_ref, g_ref, b_ref, w_ref, gn_ref):
        dis = _dis_from(degp_ref[...])
        s = jnp.concatenate([p_ref[0] + g_ref[0], p_ref[1] + g_ref[1]],
                            axis=-1)
        o = dis[:, None] * s + b_ref[...]
        a = jnp.where(o > 0, o, jnp.exp(o) - 1.0)
        h = jnp.dot(a, w_ref[...], preferred_element_type=jnp.float32)
        gn_ref[...] = _split(dis[:, None] * h)

    return pl.pallas_call(
        body,
        grid=(n // _B,),
        in_specs=[
            pl.BlockSpec((NC, _B, L), lambda i: (0, i, 0)),
            pl.BlockSpec((NC, _B, dh), lambda i: (0, i, 0)),
            pl.BlockSpec((NC, _B, dh), lambda i: (0, i, 0)),
            pl.BlockSpec((1, d), lambda i: (0, 0)),
            pl.BlockSpec((d, 2 * dhn), lambda i: (0, 0)),
        ],
        out_specs=pl.BlockSpec((NC, _B, dhn), lambda i: (0, i, 0)),
        out_shape=jax.ShapeDtypeStruct((NC, n, dhn), jnp.float32),
    )(degp, p, gh, b.reshape(1, d), w)


def _tc_final(degp, p, gh, b):
    n = gh.shape[1]
    dh = gh.shape[2]
    d = 2 * dh

    def body(degp_ref, p_ref, g_ref, b_ref, o_ref):
        dis = _dis_from(degp_ref[...])
        s = jnp.concatenate([p_ref[0] + g_ref[0], p_ref[1] + g_ref[1]],
                            axis=-1)
        o_ref[...] = dis[:, None] * s + b_ref[...]

    return pl.pallas_call(
        body,
        grid=(n // _B,),
        in_specs=[
            pl.BlockSpec((NC, _B, L), lambda i: (0, i, 0)),
            pl.BlockSpec((NC, _B, dh), lambda i: (0, i, 0)),
            pl.BlockSpec((NC, _B, dh), lambda i: (0, i, 0)),
            pl.BlockSpec((1, d), lambda i: (0, 0)),
        ],
        out_specs=pl.BlockSpec((_B, d), lambda i: (i, 0)),
        out_shape=jax.ShapeDtypeStruct((n, d), jnp.float32),
    )(degp, p, gh, b.reshape(1, d))


def kernel(x, edge_index, edge_attr, W1, b1, W2, b2, W3, b3):
    n = x.shape[0]
    e = edge_index.shape[1]
    c_chunks = -(-e // (NS * CH * NBUF * NC)) * NBUF * NC
    e_pad = NS * CH * c_chunks
    pad = e_pad - e

    row3 = jnp.pad(edge_index[0], (0, pad)).reshape(NS, c_chunks, CH)
    col3 = jnp.pad(edge_index[1], (0, pad)).reshape(NS, c_chunks, CH)
    ew = jnp.pad(edge_attr, (0, pad)).reshape(NS, c_chunks, CH)
    ewb = jnp.broadcast_to(ew[..., None], (NS, c_chunks, CH, L)) + 0.0

    degp = _sc_degree(col3, ewb, n)
    g1 = _tc_pre(degp, x, W1)
    p1 = _sc_scatter(g1, row3, col3, ewb, n)
    g2 = _tc_mid(degp, p1, g1, b1, W2)
    p2 = _sc_scatter(g2, row3, col3, ewb, n)
    g3 = _tc_mid(degp, p2, g2, b2, W3)
    p3 = _sc_scatter(g3, row3, col3, ewb, n)
    return _tc_final(degp, p3, g3, b3)


# speedup vs baseline: 1.2165x; 1.2165x over previous
import functools

import jax
import jax.numpy as jnp
from jax import lax
from jax.experimental import pallas as pl
from jax.experimental.pallas import tpu as pltpu
from jax.experimental.pallas import tpu_sc as plsc

NC = 2
NS = 16
L = 16
CH = 128
AHEAD = 2
NBUF = 4


def _mesh():
    return plsc.VectorSubcoreMesh(core_axis_name="c", subcore_axis_name="s")


_SC_PARAMS = pltpu.CompilerParams(use_tc_tiling_on_sc=False,
                                  needs_layout_passes=False)


def _pad_nodes(n_nodes):
    per = -(-n_nodes // (NS * 40)) * 40
    return per * NS


def _zero_acc_slice(zb, acc, sid, n_rows_per_subcore):
    zr, d = zb.shape

    @pl.loop(0, zr)
    def _(r):
        for f in range(d // L):
            zb[r, pl.ds(f * L, L)] = jnp.zeros((L,), jnp.float32)

    @pl.loop(0, n_rows_per_subcore // zr)
    def _(z):
        pltpu.sync_copy(zb, acc.at[pl.ds(sid * n_rows_per_subcore + z * zr, zr)])


def _splat(ref1d, j):
    return plsc.load_gather(ref1d, [jnp.full((L,), j, jnp.int32)])


def _sc_degree(col3, ew3, n_nodes):
    c_chunks = col3.shape[1]
    c_half = c_chunks // NC
    n_pad = _pad_nodes(n_nodes)
    rps = n_pad // NS

    @functools.partial(
        pl.kernel,
        out_type=jax.ShapeDtypeStruct((NC, n_pad, L), jnp.float32),
        mesh=_mesh(),
        compiler_params=_SC_PARAMS,
        scratch_types=[
            pltpu.VMEM((c_chunks, CH), jnp.int32),
            pltpu.VMEM((NBUF, CH), jnp.float32),
            pltpu.VMEM((NBUF, CH, L), jnp.float32),
            pltpu.VMEM_SHARED((n_pad, L), jnp.float32),
            pltpu.SemaphoreType.DMA((NBUF,)),
            pltpu.SemaphoreType.DMA((NBUF,)),
        ],
    )
    def k(col_hbm, ew_hbm, out_hbm, col_v, ewv, bb, acc, esem, ssem):
        cid = lax.axis_index("c")
        sid = lax.axis_index("s")
        base = cid * c_half
        pltpu.sync_copy(col_hbm.at[sid], col_v)
        _zero_acc_slice(bb.at[0], acc, sid, rps)
        plsc.subcore_barrier()

        def issue(ci, b):
            pltpu.async_copy(ew_hbm.at[sid, base + ci], ewv.at[b],
                             esem.at[b])

        def wait_scatter(b):
            pltpu.make_async_copy(bb.at[b], acc.at[col_v.at[0]],
                                  ssem.at[b]).wait()

        issue(0, 0)
        issue(1, 1)

        @pl.loop(0, c_half // NBUF)
        def _(c4):
            for kk in range(NBUF):
                b = kk
                f = (kk + AHEAD) % NBUF
                c = c4 * NBUF + kk

                @pl.when(c + AHEAD < c_half)
                def _():
                    @pl.when(c + AHEAD >= NBUF)
                    def _():
                        wait_scatter(f)

                    issue(c + AHEAD, f)

                pltpu.make_async_copy(ew_hbm.at[sid, 0], ewv.at[b],
                                      esem.at[b]).wait()

                evb = ewv.at[b]
                bbb = bb.at[b]

                @plsc.parallel_loop(0, CH, step=L, unroll=2)
                def _(q):
                    for jj in range(L):
                        bbb[q + jj, :] = _splat(evb, q + jj)

                pltpu.async_copy(bb.at[b], acc.at[col_v.at[base + c]],
                                 ssem.at[b], add=True)

        for b in range(NBUF):
            wait_scatter(b)

        plsc.subcore_barrier()
        pltpu.sync_copy(acc.at[pl.ds(sid * rps, rps)],
                        out_hbm.at[cid, pl.ds(sid * rps, rps)])

    return k(col3, ew3)


def _sc_scatter(gh, row3, col3, ew3, n_nodes):
    dh = gh.shape[2]
    c_chunks = row3.shape[1]
    n_pad = _pad_nodes(n_nodes)
    rps = n_pad // NS
    fd = dh // L

    @functools.partial(
        pl.kernel,
        out_type=jax.ShapeDtypeStruct((NC, n_pad, dh), jnp.float32),
        mesh=_mesh(),
        compiler_params=_SC_PARAMS,
        scratch_types=[
            pltpu.VMEM((c_chunks, CH), jnp.int32),
            pltpu.VMEM((c_chunks, CH), jnp.int32),
            pltpu.VMEM((NBUF, CH), jnp.float32),
            pltpu.VMEM((NBUF, CH, dh), jnp.float32),
            pltpu.VMEM_SHARED((n_pad, dh), jnp.float32),
            pltpu.SemaphoreType.DMA((NBUF,)),
            pltpu.SemaphoreType.DMA((NBUF,)),
            pltpu.SemaphoreType.DMA((NBUF,)),
        ],
    )
    def k(gh_hbm, row_hbm, col_hbm, ew_hbm, out_hbm,
          row_v, col_v, ewv, gbuf, acc, gsem, esem, ssem):
        cid = lax.axis_index("c")
        sid = lax.axis_index("s")
        g_src = gh_hbm.at[cid]

        pltpu.sync_copy(row_hbm.at[sid], row_v)
        pltpu.sync_copy(col_hbm.at[sid], col_v)
        _zero_acc_slice(gbuf.at[0], acc, sid, rps)
        plsc.subcore_barrier()

        def issue(ci, b):
            pltpu.async_copy(ew_hbm.at[sid, ci], ewv.at[b], esem.at[b])
            pltpu.async_copy(g_src.at[row_v.at[ci]], gbuf.at[b], gsem.at[b])

        def wait_scatter(b):
            pltpu.make_async_copy(gbuf.at[b], acc.at[col_v.at[0]],
                                  ssem.at[b]).wait()

        issue(0, 0)
        issue(1, 1)

        @pl.loop(0, c_chunks // NBUF)
        def _(c4):
            for kk in range(NBUF):
                b = kk
                f = (kk + AHEAD) % NBUF
                c = c4 * NBUF + kk

                @pl.when(c + AHEAD < c_chunks)
                def _():
                    @pl.when(c + AHEAD >= NBUF)
                    def _():
                        wait_scatter(f)

                    issue(c + AHEAD, f)

                pltpu.make_async_copy(g_src.at[row_v.at[0]], gbuf.at[b],
                                      gsem.at[b]).wait()
                pltpu.make_async_copy(ew_hbm.at[sid, 0], ewv.at[b],
                                      esem.at[b]).wait()

                eb = ewv.at[b]
                gb = gbuf.at[b]

                @plsc.parallel_loop(0, CH, step=L, unroll=2)
                def _(q):
                    for jj in range(L):
                        s = _splat(eb, q + jj)
                        for ff in range(fd):
                            sl = pl.ds(ff * L, L)
                            gb[q + jj, sl] = gb[q + jj, sl] * s

                pltpu.async_copy(gb, acc.at[col_v.at[c]], ssem.at[b],
                                 add=True)

        for b in range(NBUF):
            wait_scatter(b)

        plsc.subcore_barrier()
        pltpu.sync_copy(acc.at[pl.ds(sid * rps, rps)],
                        out_hbm.at[cid, pl.ds(sid * rps, rps)])

    return k(gh, row3, col3, ew3)


def _dis_from(degp_blk):
    deg = degp_blk[0, :, 0] + degp_blk[1, :, 0] + 1.0
    return jnp.where(deg > 0, lax.rsqrt(jnp.maximum(deg, 1e-30)), 0.0)


def _split(g):
    dh = g.shape[1] // 2
    return jnp.stack([g[:, :dh], g[:, dh:]], axis=0)


_B = 1000


def _tc_pre(degp, x, w):
    n, d_in = x.shape
    dh = w.shape[1] // 2

    def body(degp_ref, x_ref, w_ref, g_ref):
        dis = _dis_from(degp_ref[...])
        h = jnp.dot(x_ref[...], w_ref[...], preferred_element_type=jnp.float32)
        g_ref[...] = _split(dis[:, None] * h)

    return pl.pallas_call(
        body,
        grid=(n // _B,),
        in_specs=[
            pl.BlockSpec((NC, _B, L), lambda i: (0, i, 0)),
            pl.BlockSpec((_B, d_in), lambda i: (i, 0)),
            pl.BlockSpec((d_in, 2 * dh), lambda i: (0, 0)),
        ],
        out_specs=pl.BlockSpec((NC, _B, dh), lambda i: (0, i, 0)),
        out_shape=jax.ShapeDtypeStruct((NC, n, dh), jnp.float32),
    )(degp, x, w)


def _tc_mid(degp, p, gh, b, w):
    n = gh.shape[1]
    dh = gh.shape[2]
    d = 2 * dh
    dhn = w.shape[1] // 2

    def body(degp_ref, p_ref, g_ref, b_ref, w_ref, gn_ref):
        dis = _dis_from(degp_ref[...])
        s = jnp.concatenate([p_ref[0] + g_ref[0], p_ref[1] + g_ref[1]],
                            axis=-1)
        o = dis[:, None] * s + b_ref[...]
        a = jnp.where(o > 0, o, jnp.exp(o) - 1.0)
        h = jnp.dot(a, w_ref[...], preferred_element_type=jnp.float32)
        gn_ref[...] = _split(dis[:, None] * h)

    return pl.pallas_call(
        body,
        grid=(n // _B,),
        in_specs=[
            pl.BlockSpec((NC, _B, L), lambda i: (0, i, 0)),
            pl.BlockSpec((NC, _B, dh), lambda i: (0, i, 0)),
            pl.BlockSpec((NC, _B, dh), lambda i: (0, i, 0)),
            pl.BlockSpec((1, d), lambda i: (0, 0)),
            pl.BlockSpec((d, 2 * dhn), lambda i: (0, 0)),
        ],
        out_specs=pl.BlockSpec((NC, _B, dhn), lambda i: (0, i, 0)),
        out_shape=jax.ShapeDtypeStruct((NC, n, dhn), jnp.float32),
    )(degp, p, gh, b.reshape(1, d), w)


def _tc_final(degp, p, gh, b):
    n = gh.shape[1]
    dh = gh.shape[2]
    d = 2 * dh

    def body(degp_ref, p_ref, g_ref, b_ref, o_ref):
        dis = _dis_from(degp_ref[...])
        s = jnp.concatenate([p_ref[0] + g_ref[0], p_ref[1] + g_ref[1]],
                            axis=-1)
        o_ref[...] = dis[:, None] * s + b_ref[...]

    return pl.pallas_call(
        body,
        grid=(n // _B,),
        in_specs=[
            pl.BlockSpec((NC, _B, L), lambda i: (0, i, 0)),
            pl.BlockSpec((NC, _B, dh), lambda i: (0, i, 0)),
            pl.BlockSpec((NC, _B, dh), lambda i: (0, i, 0)),
            pl.BlockSpec((1, d), lambda i: (0, 0)),
        ],
        out_specs=pl.BlockSpec((_B, d), lambda i: (i, 0)),
        out_shape=jax.ShapeDtypeStruct((n, d), jnp.float32),
    )(degp, p, gh, b.reshape(1, d))


def kernel(x, edge_index, edge_attr, W1, b1, W2, b2, W3, b3):
    n = x.shape[0]
    e = edge_index.shape[1]
    c_chunks = -(-e // (NS * CH * NBUF * NC)) * NBUF * NC
    e_pad = NS * CH * c_chunks
    pad = e_pad - e

    row3 = jnp.pad(edge_index[0], (0, pad)).reshape(NS, c_chunks, CH)
    col3 = jnp.pad(edge_index[1], (0, pad)).reshape(NS, c_chunks, CH)
    ew3 = jnp.pad(edge_attr, (0, pad)).reshape(NS, c_chunks, CH)

    degp = _sc_degree(col3, ew3, n)
    g1 = _tc_pre(degp, x, W1)
    p1 = _sc_scatter(g1, row3, col3, ew3, n)
    g2 = _tc_mid(degp, p1, g1, b1, W2)
    p2 = _sc_scatter(g2, row3, col3, ew3, n)
    g3 = _tc_mid(degp, p2, g2, b2, W3)
    p3 = _sc_scatter(g3, row3, col3, ew3, n)
    return _tc_final(degp, p3, g3, b3)
